# baseline (speedup 1.0000x reference)
import functools

import jax
import jax.numpy as jnp
from jax import lax
from jax.experimental import pallas as pl
from jax.experimental.pallas import tpu as pltpu
from jax.experimental.pallas import tpu_sc as plsc

HIDDEN = 768
NUM_CORES = 2
NUM_SUBCORES = 16
NW = NUM_CORES * NUM_SUBCORES

NBUF = 2
CHUNK = 64


def _sc_gather(table, idx_flat, b_total):
    b_per_w = b_total // NW
    n_chunks = b_per_w // CHUNK
    mesh = plsc.VectorSubcoreMesh(core_axis_name="c", subcore_axis_name="s")

    @functools.partial(
        pl.kernel,
        mesh=mesh,
        out_type=jax.ShapeDtypeStruct((b_total, HIDDEN), jnp.float32),
        scratch_types=[
            pltpu.VMEM((b_per_w,), jnp.int32),
            *([pltpu.VMEM((CHUNK, HIDDEN), jnp.float32)] * NBUF),
            *([pltpu.SemaphoreType.DMA] * NBUF),
            *([pltpu.SemaphoreType.DMA] * NBUF),
        ],
    )
    def k(table_hbm, idx_hbm, out_hbm, idx_v, *scratch):
        bufs = scratch[:NBUF]
        gsems = scratch[NBUF : 2 * NBUF]
        osems = scratch[2 * NBUF :]
        wid = lax.axis_index("s") * NUM_CORES + lax.axis_index("c")
        base = wid * b_per_w
        pltpu.sync_copy(idx_hbm.at[pl.ds(base, b_per_w)], idx_v)

        def gather(c, b):
            pltpu.async_copy(
                table_hbm.at[idx_v.at[pl.ds(c * CHUNK, CHUNK)]],
                bufs[b],
                gsems[b],
            )

        def wait_gather(c, b):
            pltpu.make_async_copy(
                table_hbm.at[idx_v.at[pl.ds(c * CHUNK, CHUNK)]],
                bufs[b],
                gsems[b],
            ).wait()

        def put(c, b):
            pltpu.async_copy(
                bufs[b], out_hbm.at[pl.ds(base + c * CHUNK, CHUNK)], osems[b]
            )

        def wait_put(c, b):
            pltpu.make_async_copy(
                bufs[b], out_hbm.at[pl.ds(base + c * CHUNK, CHUNK)], osems[b]
            ).wait()

        for b in range(NBUF):
            gather(b, b)

        def body(i, carry):
            for b in range(NBUF):
                c = NBUF * i + b
                wait_gather(c, b)
                put(c, b)
                nxt = (b + 1) % NBUF

                @pl.when((c >= NBUF - 1) & (c + 1 < n_chunks))
                def _():
                    wait_put(c - (NBUF - 1), nxt)
                    gather(c + 1, nxt)

            return carry

        lax.fori_loop(0, n_chunks // NBUF, body, 0)

        for b in range(NBUF):
            c = n_chunks - NBUF + b
            wait_put(c, b)

    return k(table, idx_flat)


def kernel(position_ids, table):
    batch, seq = position_ids.shape
    b_total = batch * seq
    idx_flat = position_ids.reshape(b_total).astype(jnp.int32)
    out = _sc_gather(table, idx_flat, b_total)
    return out.reshape(batch, seq, HIDDEN)


# speedup vs baseline: 1.0019x; 1.0019x over previous
import functools

import jax
import jax.numpy as jnp
from jax import lax
from jax.experimental import pallas as pl
from jax.experimental.pallas import tpu as pltpu
from jax.experimental.pallas import tpu_sc as plsc

HIDDEN = 768
NUM_CORES = 2
NUM_SUBCORES = 16
NW = NUM_CORES * NUM_SUBCORES

NBUF = 2
CHUNK = 64


def _sc_gather(table, position_ids, batch, seq):
    b_per_w = (batch * seq) // NW
    n_chunks = b_per_w // CHUNK
    w_per_row = seq // b_per_w
    mesh = plsc.VectorSubcoreMesh(core_axis_name="c", subcore_axis_name="s")

    @functools.partial(
        pl.kernel,
        mesh=mesh,
        out_type=jax.ShapeDtypeStruct((batch, seq, HIDDEN), jnp.float32),
        scratch_types=[
            pltpu.VMEM((b_per_w,), jnp.int32),
            *([pltpu.VMEM((CHUNK, HIDDEN), jnp.float32)] * NBUF),
            *([pltpu.SemaphoreType.DMA] * NBUF),
            *([pltpu.SemaphoreType.DMA] * NBUF),
        ],
    )
    def k(table_hbm, idx_hbm, out_hbm, idx_v, *scratch):
        bufs = scratch[:NBUF]
        gsems = scratch[NBUF : 2 * NBUF]
        osems = scratch[2 * NBUF :]
        wid = lax.axis_index("s") * NUM_CORES + lax.axis_index("c")
        row = wid // w_per_row
        col = (wid % w_per_row) * b_per_w
        pltpu.sync_copy(idx_hbm.at[row, pl.ds(col, b_per_w)], idx_v)

        def gather(c, b):
            pltpu.async_copy(
                table_hbm.at[idx_v.at[pl.ds(c * CHUNK, CHUNK)]],
                bufs[b],
                gsems[b],
            )

        def wait_gather(c, b):
            pltpu.make_async_copy(
                table_hbm.at[idx_v.at[pl.ds(c * CHUNK, CHUNK)]],
                bufs[b],
                gsems[b],
            ).wait()

        def put(c, b):
            pltpu.async_copy(
                bufs[b],
                out_hbm.at[row, pl.ds(col + c * CHUNK, CHUNK)],
                osems[b],
            )

        def wait_put(c, b):
            pltpu.make_async_copy(
                bufs[b],
                out_hbm.at[row, pl.ds(col + c * CHUNK, CHUNK)],
                osems[b],
            ).wait()

        for b in range(NBUF):
            gather(b, b)

        def body(i, carry):
            for b in range(NBUF):
                c = NBUF * i + b
                wait_gather(c, b)
                put(c, b)
                nxt = (b + 1) % NBUF

                @pl.when((c >= NBUF - 1) & (c + 1 < n_chunks))
                def _():
                    wait_put(c - (NBUF - 1), nxt)
                    gather(c + 1, nxt)

            return carry

        lax.fori_loop(0, n_chunks // NBUF, body, 0)

        for b in range(NBUF):
            c = n_chunks - NBUF + b
            wait_put(c, b)

    return k(table, position_ids)


def kernel(position_ids, table):
    batch, seq = position_ids.shape
    return _sc_gather(table, position_ids.astype(jnp.int32), batch, seq)
